# baseline (speedup 1.0000x reference)
import functools

import jax
import jax.numpy as jnp
from jax import lax
from jax.experimental import pallas as pl
from jax.experimental.pallas import tpu as pltpu
from jax.experimental.pallas import tpu_sc as plsc

N, E, NF, HID, NC, NL, NG = 10000, 320000, 128, 128, 10, 3, 64
BN_EPS = 1e-5

BLK = 1000
NBLK = N // BLK

NCORE, NSUB = 2, 16
NWORK = NCORE * NSUB
CH = 128
KPB = 8
EDGE_BLK = CH * KPB
NB = 10
EPW = NB * EDGE_BLK
E_PAD = EPW * NWORK
PAD_ROWS = 16
NPAD = N + PAD_ROWS
RCH = 128
NRCH_FULL = N // RCH
RTAIL = N - NRCH_FULL * RCH
RTAIL_OFF = NRCH_FULL * RCH
NRCH_PER_TILE = (NRCH_FULL + 1 + NSUB - 1) // NSUB


def _agg_body(h_hbm, srcb, dstb, agg0, agg1, src_i, dst_i, rows, acc, gsem):
    cid = lax.axis_index("c")
    sid = lax.axis_index("s")
    wid = sid * NCORE + cid

    for k in range(NRCH_PER_TILE):
        c = sid + NSUB * k

        @pl.when(c < NRCH_FULL)
        def _():
            off = c * RCH
            pltpu.sync_copy(h_hbm.at[pl.ds(off, RCH)],
                            acc.at[pl.ds(off, RCH)])

        @pl.when(c == NRCH_FULL)
        def _():
            pltpu.sync_copy(h_hbm.at[pl.ds(RTAIL_OFF, RTAIL)],
                            acc.at[pl.ds(RTAIL_OFF, RTAIL)])

    plsc.subcore_barrier()

    def blk_body(blk, carry):
        bid = (blk // NB) * NCORE * NB + cid * NB + (blk % NB)
        pltpu.sync_copy(srcb.at[bid], src_i)
        pltpu.sync_copy(dstb.at[bid], dst_i)
        for c in range(KPB):
            pltpu.async_copy(h_hbm.at[src_i.at[c]], rows, gsem).wait()
            pltpu.sync_copy(rows, acc.at[dst_i.at[c]], add=True)
        return carry

    @pl.when(sid == 0)
    def _():
        lax.fori_loop(0, NB * NSUB, blk_body, 0)
    plsc.subcore_barrier()

    for k in range(NRCH_PER_TILE):
        c = sid + NSUB * k

        @pl.when(c < NRCH_FULL)
        def _():
            off = c * RCH

            @pl.when(cid == 0)
            def _():
                pltpu.sync_copy(acc.at[pl.ds(off, RCH)],
                                agg0.at[pl.ds(off, RCH)])

            @pl.when(cid == 1)
            def _():
                pltpu.sync_copy(acc.at[pl.ds(off, RCH)],
                                agg1.at[pl.ds(off, RCH)])

        @pl.when(c == NRCH_FULL)
        def _():
            @pl.when(cid == 0)
            def _():
                pltpu.sync_copy(acc.at[pl.ds(RTAIL_OFF, RTAIL)],
                                agg0.at[pl.ds(RTAIL_OFF, RTAIL)])

            @pl.when(cid == 1)
            def _():
                pltpu.sync_copy(acc.at[pl.ds(RTAIL_OFF, RTAIL)],
                                agg1.at[pl.ds(RTAIL_OFF, RTAIL)])


@functools.cache
def _make_agg():
    return pl.kernel(
        _agg_body,
        out_type=[
            jax.ShapeDtypeStruct((N, HID), jnp.float32),
            jax.ShapeDtypeStruct((N, HID), jnp.float32),
        ],
        mesh=plsc.VectorSubcoreMesh(core_axis_name="c", subcore_axis_name="s"),
        scratch_types=[
            pltpu.VMEM((KPB, CH), jnp.int32),
            pltpu.VMEM((KPB, CH), jnp.int32),
            pltpu.VMEM((CH, HID), jnp.float32),
            pltpu.VMEM_SHARED((NPAD, HID), jnp.float32),
            pltpu.SemaphoreType.DMA,
        ],
    )


def _dot(a, b):
    return jnp.dot(a, b, preferred_element_type=jnp.float32,
                   precision=lax.Precision.HIGHEST)


def _mlp_block(z, w1, b1, w2, b2, g, be):
    z = jnp.maximum(_dot(z, w1) + b1, 0.0)
    z = jnp.maximum(_dot(z, w2) + b2, 0.0)
    return z * g + be


def _layer_body(h_ref, a0_ref, a1_ref, w1_ref, b1_ref, w2_ref, b2_ref,
                g_ref, be_ref, o_ref):
    z = a0_ref[...] + a1_ref[...] - h_ref[...]
    o_ref[...] = _mlp_block(z, w1_ref[...], b1_ref[...], w2_ref[...],
                            b2_ref[...], g_ref[...], be_ref[...])


def _row_spec():
    return pl.BlockSpec((BLK, HID), lambda i: (i, 0))


def _full_spec(shape):
    return pl.BlockSpec(shape, lambda i: (0,) * len(shape))


_layer_call = pl.pallas_call(
    _layer_body,
    grid=(NBLK,),
    in_specs=[_row_spec(), _row_spec(), _row_spec(),
              _full_spec((HID, HID)), _full_spec((1, HID)),
              _full_spec((HID, HID)), _full_spec((1, HID)),
              _full_spec((1, HID)), _full_spec((1, HID))],
    out_specs=_row_spec(),
    out_shape=jax.ShapeDtypeStruct((N, HID), jnp.float32),
)


def _final_body(h_ref, a0_ref, a1_ref, w1_ref, b1_ref, w2_ref, b2_ref,
                g_ref, be_ref, batch_ref, l1w_ref, l1b_ref, l2w_ref, l2b_ref,
                emb_ref, logp_ref, out_ref, pool_acc, cnt_acc):
    i = pl.program_id(0)

    @pl.when(i == 0)
    def _():
        pool_acc[...] = jnp.zeros_like(pool_acc)
        cnt_acc[...] = jnp.zeros_like(cnt_acc)

    z = a0_ref[...] + a1_ref[...] - h_ref[...]
    h3 = _mlp_block(z, w1_ref[...], b1_ref[...], w2_ref[...], b2_ref[...],
                    g_ref[...], be_ref[...])
    emb_ref[...] = h3

    bt = batch_ref[...].reshape(1, BLK)
    seg = lax.broadcasted_iota(jnp.int32, (NG, BLK), 0)
    mask = (bt == seg).astype(jnp.float32)
    pool_acc[...] += lax.dot_general(
        mask, h3, (((1,), (0,)), ((), ())),
        preferred_element_type=jnp.float32, precision=lax.Precision.HIGHEST)
    cnt_acc[...] += jnp.broadcast_to(
        jnp.sum(mask, axis=1, keepdims=True), (NG, HID))

    @pl.when(i == NBLK - 1)
    def _():
        pooled = pool_acc[...] / jnp.maximum(cnt_acc[...], 1.0)
        z1 = _dot(pooled, l1w_ref[...]) + l1b_ref[...]
        z2 = _dot(z1, l2w_ref[...]) + l2b_ref[...]
        outv = jnp.clip(z2, -10.0, 10.0)
        out_ref[...] = outv
        lane = lax.broadcasted_iota(jnp.int32, (NG, HID), 1)
        masked = jnp.where(lane < NC, outv, -1e30)
        m = jnp.max(masked, axis=1, keepdims=True)
        lse = jnp.log(jnp.sum(jnp.exp(masked - m), axis=1, keepdims=True)) + m
        logp_ref[...] = outv - lse


_final_call = pl.pallas_call(
    _final_body,
    grid=(NBLK,),
    in_specs=[_row_spec(), _row_spec(), _row_spec(),
              _full_spec((HID, HID)), _full_spec((1, HID)),
              _full_spec((HID, HID)), _full_spec((1, HID)),
              _full_spec((1, HID)), _full_spec((1, HID)),
              pl.BlockSpec((1, 1, BLK), lambda i: (i, 0, 0)),
              _full_spec((HID, HID)), _full_spec((1, HID)),
              _full_spec((HID, HID)), _full_spec((1, HID))],
    out_specs=[_row_spec(), _full_spec((NG, HID)), _full_spec((NG, HID))],
    out_shape=[jax.ShapeDtypeStruct((N, HID), jnp.float32),
               jax.ShapeDtypeStruct((NG, HID), jnp.float32),
               jax.ShapeDtypeStruct((NG, HID), jnp.float32)],
    scratch_shapes=[pltpu.VMEM((NG, HID), jnp.float32),
                    pltpu.VMEM((NG, HID), jnp.float32)],
)

_BN_INV = 1.0 / (1.0 + BN_EPS) ** 0.5


def kernel(x, edge_index, batch, params):
    src = edge_index[0]
    dst = edge_index[1]
    npad = E_PAD - E
    pad_idx = jnp.arange(npad, dtype=jnp.int32)
    srcb = jnp.concatenate([src, pad_idx % N]).reshape(NWORK * NB, KPB, CH)
    dstb = jnp.concatenate([dst, N + (pad_idx % PAD_ROWS)]).reshape(
        NWORK * NB, KPB, CH)
    batch3 = batch.reshape(NBLK, 1, BLK)

    def layer_args(p):
        return (p['W1'], p['b1'].reshape(1, HID), p['W2'],
                p['b2'].reshape(1, HID),
                (p['gamma'] * _BN_INV).reshape(1, HID),
                p['beta'].reshape(1, HID))

    h = x
    for i in range(NL - 1):
        a0, a1 = _make_agg()(h, srcb, dstb)
        h = _layer_call(h, a0, a1, *layer_args(params['c%d' % i]))

    a0, a1 = _make_agg()(h, srcb, dstb)
    l2w = jnp.zeros((HID, HID), jnp.float32).at[:, :NC].set(params['lin2_W'])
    l2b = jnp.zeros((1, HID), jnp.float32).at[0, :NC].set(params['lin2_b'])
    embeds, logp128, out128 = _final_call(
        h, a0, a1, *layer_args(params['c%d' % (NL - 1)]),
        batch3, params['lin1_W'], params['lin1_b'].reshape(1, HID), l2w, l2b)
    return (logp128[:, :NC], embeds, out128[:, :NC])
